# Initial kernel scaffold; baseline (speedup 1.0000x reference)
import functools

import jax
import jax.numpy as jnp
from jax.experimental import pallas as pl
from jax.experimental.pallas import tpu as pltpu

N_NODES = 10000
N_GRAPHS = 128
NEG_SLOPE = 0.2

_BLK = 1000


def _mm_attn_body(x_ref, w_ref, apad_ref, h_ref, al_ref):
    h = jnp.dot(x_ref[...], w_ref[...], preferred_element_type=jnp.float32)
    h_ref[...] = h
    al_ref[...] = jnp.dot(h, apad_ref[...], preferred_element_type=jnp.float32)


def _mm_attn(x, W, a_src, a_dst):
    n, d_in = x.shape
    d_out = W.shape[1]
    apad = jnp.zeros((d_out, 128), jnp.float32)
    apad = apad.at[:, 0].set(a_src).at[:, 1].set(a_dst)
    h, al = pl.pallas_call(
        _mm_attn_body,
        grid=(n // _BLK,),
        in_specs=[
            pl.BlockSpec((_BLK, d_in), lambda i: (i, 0)),
            pl.BlockSpec((d_in, d_out), lambda i: (0, 0)),
            pl.BlockSpec((d_out, 128), lambda i: (0, 0)),
        ],
        out_specs=[
            pl.BlockSpec((_BLK, d_out), lambda i: (i, 0)),
            pl.BlockSpec((_BLK, 128), lambda i: (i, 0)),
        ],
        out_shape=[
            jax.ShapeDtypeStruct((n, d_out), jnp.float32),
            jax.ShapeDtypeStruct((n, 128), jnp.float32),
        ],
    )(x, W, apad)
    return h, al[:, 0], al[:, 1]


def _readout_body(h_ref, batch_ref, pw_ref, pb_ref, rw_ref, rb_ref, out_ref):
    b = batch_ref[...]
    onehot = (b == jax.lax.broadcasted_iota(jnp.int32, (1, N_GRAPHS), 1)).astype(jnp.float32)
    g = jnp.dot(onehot.T, h_ref[...], preferred_element_type=jnp.float32)
    g = jnp.maximum(jnp.dot(g, pw_ref[...], preferred_element_type=jnp.float32) + pb_ref[...], 0.0)
    logits = jnp.dot(g, rw_ref[...], preferred_element_type=jnp.float32) + rb_ref[...]
    m = jnp.max(logits, axis=1, keepdims=True)
    s = logits - m
    lse = jnp.log(jnp.sum(jnp.exp(s), axis=1, keepdims=True))
    out_ref[...] = s - lse


def _readout(h, batch, postW, postb, roW, rob):
    n, d = h.shape
    n_class = roW.shape[1]
    ro_pad = jnp.zeros((d, 128), jnp.float32).at[:, :n_class].set(roW)
    rb_pad = jnp.zeros((128,), jnp.float32).at[:n_class].set(rob)
    out = pl.pallas_call(
        _readout_body,
        in_specs=[
            pl.BlockSpec((n, d), lambda: (0, 0)),
            pl.BlockSpec((n, 1), lambda: (0, 0)),
            pl.BlockSpec((d, d), lambda: (0, 0)),
            pl.BlockSpec((1, d), lambda: (0, 0)),
            pl.BlockSpec((d, 128), lambda: (0, 0)),
            pl.BlockSpec((1, 128), lambda: (0, 0)),
        ],
        out_specs=pl.BlockSpec((N_GRAPHS, 128), lambda: (0, 0)),
        out_shape=jax.ShapeDtypeStruct((N_GRAPHS, 128), jnp.float32),
    )(h, batch[:, None], postW, postb[None, :], ro_pad, rb_pad[None, :])
    return out[:, :n_class]


def _edge_phase(h, alpha_s, alpha_d, src, dst, b):
    n = h.shape[0]
    e = alpha_s[src] + alpha_d[dst]
    e = jnp.where(e > 0, e, NEG_SLOPE * e)
    e_max = jax.ops.segment_max(e, dst, num_segments=n)
    e_max = jnp.where(jnp.isfinite(e_max), e_max, 0.0)
    ex = jnp.exp(e - e_max[dst])
    denom = jax.ops.segment_sum(ex, dst, num_segments=n)
    alpha = ex / (denom[dst] + 1e-16)
    out = jax.ops.segment_sum(h[src] * alpha[:, None], dst, num_segments=n)
    return out + b


def kernel(x, edge_index, batch, W1, a_src1, a_dst1, b1, W2, a_src2, a_dst2, b2,
           W3, a_src3, a_dst3, b3, postW, postb, roW, rob):
    n = x.shape[0]
    loops = jnp.arange(n, dtype=edge_index.dtype)
    src = jnp.concatenate([edge_index[0], loops])
    dst = jnp.concatenate([edge_index[1], loops])

    h = x
    for (W, a_s, a_d, b) in ((W1, a_src1, a_dst1, b1), (W2, a_src2, a_dst2, b2),
                             (W3, a_src3, a_dst3, b3)):
        hw, al_s, al_d = _mm_attn(h, W, a_s, a_d)
        h = jax.nn.relu(_edge_phase(hw, al_s, al_d, src, dst, b))

    return _readout(h, batch, postW, postb, roW, rob)


# speedup vs baseline: 1.1061x; 1.1061x over previous
import functools

import jax
import jax.numpy as jnp
from jax.experimental import pallas as pl
from jax.experimental.pallas import tpu as pltpu

N_NODES = 10000
N_GRAPHS = 128
N_CLASS = 10
NEG_SLOPE = 0.2

_BLK = 1000


def _mm_attn_body(x_ref, w_ref, apad_ref, h_ref, al_ref):
    h = jnp.dot(x_ref[...], w_ref[...], preferred_element_type=jnp.float32)
    h_ref[...] = h
    al_ref[...] = jnp.dot(h, apad_ref[...], preferred_element_type=jnp.float32)


def _mm_attn(x, W, a_src, a_dst):
    n, d_in = x.shape
    d_out = W.shape[1]
    apad = jnp.zeros((d_out, 128), jnp.float32)
    apad = apad.at[:, 0].set(a_src).at[:, 1].set(a_dst)
    h, al = pl.pallas_call(
        _mm_attn_body,
        grid=(n // _BLK,),
        in_specs=[
            pl.BlockSpec((_BLK, d_in), lambda i: (i, 0)),
            pl.BlockSpec((d_in, d_out), lambda i: (0, 0)),
            pl.BlockSpec((d_out, 128), lambda i: (0, 0)),
        ],
        out_specs=[
            pl.BlockSpec((_BLK, d_out), lambda i: (i, 0)),
            pl.BlockSpec((_BLK, 128), lambda i: (i, 0)),
        ],
        out_shape=[
            jax.ShapeDtypeStruct((n, d_out), jnp.float32),
            jax.ShapeDtypeStruct((n, 128), jnp.float32),
        ],
    )(x, W, apad)
    return h, al[:, 0], al[:, 1]


def _readout_body(h_ref, batch_ref, pw_ref, pb_ref, rw_ref, rb_ref, out_ref):
    b = batch_ref[...]
    onehot = (b == jax.lax.broadcasted_iota(jnp.int32, (1, N_GRAPHS), 1)).astype(jnp.float32)
    g = jnp.dot(onehot.T, h_ref[...], preferred_element_type=jnp.float32)
    g = jnp.maximum(jnp.dot(g, pw_ref[...], preferred_element_type=jnp.float32) + pb_ref[...], 0.0)
    logits = jnp.dot(g, rw_ref[...], preferred_element_type=jnp.float32) + rb_ref[...]
    col = jax.lax.broadcasted_iota(jnp.int32, logits.shape, 1)
    logits = jnp.where(col < N_CLASS, logits, -jnp.inf)
    m = jnp.max(logits, axis=1, keepdims=True)
    s = logits - m
    lse = jnp.log(jnp.sum(jnp.exp(s), axis=1, keepdims=True))
    out_ref[...] = s - lse


def _readout(h, batch, postW, postb, roW, rob):
    n, d = h.shape
    n_class = roW.shape[1]
    ro_pad = jnp.zeros((d, 128), jnp.float32).at[:, :n_class].set(roW)
    rb_pad = jnp.zeros((128,), jnp.float32).at[:n_class].set(rob)
    out = pl.pallas_call(
        _readout_body,
        in_specs=[
            pl.BlockSpec((n, d), lambda: (0, 0)),
            pl.BlockSpec((n, 1), lambda: (0, 0)),
            pl.BlockSpec((d, d), lambda: (0, 0)),
            pl.BlockSpec((1, d), lambda: (0, 0)),
            pl.BlockSpec((d, 128), lambda: (0, 0)),
            pl.BlockSpec((1, 128), lambda: (0, 0)),
        ],
        out_specs=pl.BlockSpec((N_GRAPHS, 128), lambda: (0, 0)),
        out_shape=jax.ShapeDtypeStruct((N_GRAPHS, 128), jnp.float32),
    )(h, batch[:, None], postW, postb[None, :], ro_pad, rb_pad[None, :])
    return out[:, :n_class]


def _edge_phase(h, alpha_s, alpha_d, src, dst, b):
    n = h.shape[0]
    e = alpha_s[src] + alpha_d[dst]
    e = jnp.where(e > 0, e, NEG_SLOPE * e)
    e_max = jax.ops.segment_max(e, dst, num_segments=n)
    e_max = jnp.where(jnp.isfinite(e_max), e_max, 0.0)
    ex = jnp.exp(e - e_max[dst])
    denom = jax.ops.segment_sum(ex, dst, num_segments=n)
    alpha = ex / (denom[dst] + 1e-16)
    out = jax.ops.segment_sum(h[src] * alpha[:, None], dst, num_segments=n)
    return out + b


def kernel(x, edge_index, batch, W1, a_src1, a_dst1, b1, W2, a_src2, a_dst2, b2,
           W3, a_src3, a_dst3, b3, postW, postb, roW, rob):
    n = x.shape[0]
    loops = jnp.arange(n, dtype=edge_index.dtype)
    src = jnp.concatenate([edge_index[0], loops])
    dst = jnp.concatenate([edge_index[1], loops])

    h = x
    for (W, a_s, a_d, b) in ((W1, a_src1, a_dst1, b1), (W2, a_src2, a_dst2, b2),
                             (W3, a_src3, a_dst3, b3)):
        hw, al_s, al_d = _mm_attn(h, W, a_s, a_d)
        h = jax.nn.relu(_edge_phase(hw, al_s, al_d, src, dst, b))

    return _readout(h, batch, postW, postb, roW, rob)


# speedup vs baseline: 4.1754x; 3.7749x over previous
import functools

import jax
import jax.numpy as jnp
from jax import lax
from jax.experimental import pallas as pl
from jax.experimental.pallas import tpu as pltpu
from jax.experimental.pallas import tpu_sc as plsc

N_NODES = 10000
N_PAD = 10240
NHALF = 5120
N_GRAPHS = 128
N_CLASS = 10
NEG_SLOPE = 0.2
D_HID = 256
DH = 128
N_EDGES_TOT = 330000
CHUNK = 64
EDGES_PER_TILE = 20736
E_PAD = 16 * EDGES_PER_TILE
NCHUNK = EDGES_PER_TILE // CHUNK

_BLK = 1024


def _mm_attn_body(x_ref, w_ref, apad_ref, hs_ref, al_ref):
    h = jnp.dot(x_ref[...], w_ref[...], preferred_element_type=jnp.float32)
    hs_ref[0] = h[:, :DH]
    hs_ref[1] = h[:, DH:]
    al_ref[...] = jnp.dot(h, apad_ref[...], preferred_element_type=jnp.float32)


def _mm_attn(x, W, a_src, a_dst):
    n, d_in = x.shape
    apad = jnp.zeros((D_HID, 128), jnp.float32)
    apad = apad.at[:, 0].set(a_src).at[:, 1].set(a_dst)
    hs, al = pl.pallas_call(
        _mm_attn_body,
        grid=(n // _BLK,),
        in_specs=[
            pl.BlockSpec((_BLK, d_in), lambda i: (i, 0)),
            pl.BlockSpec((d_in, D_HID), lambda i: (0, 0)),
            pl.BlockSpec((D_HID, 128), lambda i: (0, 0)),
        ],
        out_specs=[
            pl.BlockSpec((2, _BLK, DH), lambda i: (0, i, 0)),
            pl.BlockSpec((_BLK, 128), lambda i: (i, 0)),
        ],
        out_shape=[
            jax.ShapeDtypeStruct((2, n, DH), jnp.float32),
            jax.ShapeDtypeStruct((n, 128), jnp.float32),
        ],
    )(x, W, apad)
    return hs.reshape(2 * n, DH), al


EPT_A = E_PAD // 32
NCH_A = EPT_A // CHUNK


def _score_body(srcp_hbm, dstp_hbm, asv_hbm, adv_hbm, mrep_hbm,
                ex_hbm, dps_hbm,
                astab, adtab, sbuf2, dbuf2, ebuf, exw, dp_v, mv):
    core = lax.axis_index("c")
    wid = lax.axis_index("s")
    iota = lax.iota(jnp.int32, 16)

    pltpu.sync_copy(asv_hbm, astab)
    pltpu.sync_copy(adv_hbm, adtab)
    pltpu.sync_copy(mrep_hbm, mv)
    m_s = mv[...][0]

    for i in range(640):
        dp_v[i, :] = jnp.zeros((16,), jnp.float32)

    tid = core * 16 + wid
    ebase = tid * EPT_A

    def chunk_body(ch, carry):
        base = ebase + ch * CHUNK
        pltpu.sync_copy(srcp_hbm.at[pl.ds(base, CHUNK)], sbuf2.at[pl.ds(0, CHUNK)])
        pltpu.sync_copy(dstp_hbm.at[pl.ds(base, CHUNK)], dbuf2.at[pl.ds(0, CHUNK)])

        def l1(j, c):
            s = sbuf2[pl.ds(j, 16)][0]
            d = dbuf2[pl.ds(j, 16)][0]
            e = astab[pl.ds(s, 16)][0] + adtab[pl.ds(d, 16)][0]
            e = jnp.where(e > 0, e, e * NEG_SLOPE) - m_s
            ebuf[j >> 4, :] = jnp.where(iota == (j & 15), e, ebuf[j >> 4, :])
            return c
        lax.fori_loop(0, CHUNK, l1, 0, unroll=False)
        for i in range(CHUNK // 16):
            exw[pl.ds(i * 16, 16)] = jnp.exp(ebuf[i, :])

        def l2(j, c):
            exj = exw[pl.ds(j, 16)][0]
            dj = dbuf2[pl.ds(j, 16)][0]
            dp_v[dj >> 4, :] = dp_v[dj >> 4, :] + jnp.where(iota == (dj & 15), exj, 0.0)
            return c
        lax.fori_loop(0, CHUNK, l2, 0, unroll=False)

        pltpu.sync_copy(exw.at[pl.ds(0, CHUNK)], ex_hbm.at[pl.ds(base, CHUNK)])
        return carry

    lax.fori_loop(0, NCH_A, chunk_body, 0, unroll=False)
    pltpu.sync_copy(dp_v, dps_hbm.at[core, wid])


def _score_sc(srcp, dstp, asv, adv, mrep):
    mesh = plsc.VectorSubcoreMesh(core_axis_name="c", subcore_axis_name="s")
    kern = functools.partial(
        pl.kernel,
        mesh=mesh,
        out_type=(jax.ShapeDtypeStruct((E_PAD,), jnp.float32),
                  jax.ShapeDtypeStruct((2, 16, 640, 16), jnp.float32)),
        scratch_types=[
            pltpu.VMEM((N_PAD + 16,), jnp.float32),
            pltpu.VMEM((N_PAD + 16,), jnp.float32),
            pltpu.VMEM((CHUNK + 16,), jnp.int32),
            pltpu.VMEM((CHUNK + 16,), jnp.int32),
            pltpu.VMEM((CHUNK // 16, 16), jnp.float32),
            pltpu.VMEM((CHUNK + 16,), jnp.float32),
            pltpu.VMEM((640, 16), jnp.float32),
            pltpu.VMEM((16,), jnp.float32),
        ],
    )(_score_body)
    return kern(srcp, dstp, asv, adv, mrep)


def _accum_body(hs_hbm, srcp_hbm, dstp_hbm, ex_hbm, dps_hbm, bias_hbm,
                out_hbm,
                sbuf2, sidx, dbuf2, didx, exbuf, rows_v, epi_v,
                dtmp, dacc, dflat, bias_v, gsem,
                acc_sh):
    core = lax.axis_index("c")
    wid = lax.axis_index("s")

    pltpu.sync_copy(bias_hbm.at[core], bias_v)

    for i in range(40):
        dacc[i, :] = jnp.zeros((16,), jnp.float32)
    for c2 in range(2):
        for p in range(16):
            for h2 in range(2):
                start = h2 * 320 + wid * 20
                off = start & 7
                a0 = pl.multiple_of(start - off, 8)
                pltpu.sync_copy(dps_hbm.at[c2, p, pl.ds(a0, 24)], dtmp)
                for i in range(20):
                    dacc[h2 * 20 + i, :] = dacc[h2 * 20 + i, :] + dtmp[off + i, :]
    one = jnp.ones((16,), jnp.float32)
    for i in range(40):
        dflat[pl.ds(i * 16, 16)] = one / (dacc[i, :] + 1e-16)

    ebase = wid * EDGES_PER_TILE
    node0 = wid * 320

    for half in range(2):
        hbase = half * NHALF

        for k in range(DH // 16):
            epi_v[:, pl.ds(k * 16, 16)] = jnp.zeros((32, 16), jnp.float32)
        for sb in range(10):
            pltpu.sync_copy(epi_v, acc_sh.at[pl.ds(node0 + sb * 32, 32)])
        @pl.when(wid == 0)
        def _():
            pltpu.sync_copy(epi_v.at[pl.ds(0, 8)], acc_sh.at[pl.ds(NHALF, 8)])
        plsc.subcore_barrier()

        def chunk_body(ch, carry):
            base = ebase + ch * CHUNK
            pltpu.sync_copy(srcp_hbm.at[pl.ds(base, CHUNK)], sbuf2.at[pl.ds(0, CHUNK)])
            pltpu.sync_copy(dstp_hbm.at[pl.ds(base, CHUNK)], dbuf2.at[pl.ds(0, CHUNK)])
            pltpu.sync_copy(ex_hbm.at[pl.ds(base, CHUNK)], exbuf.at[pl.ds(0, CHUNK)])
            off = core * N_PAD
            for k in range(CHUNK // 16):
                sidx[pl.ds(k * 16, 16)] = sbuf2[pl.ds(k * 16, 16)] + off
            gcopy = pltpu.async_copy(hs_hbm.at[sidx], rows_v, gsem)
            for k in range(CHUNK // 16):
                v = dbuf2[pl.ds(k * 16, 16)] - hbase
                inh = (v >= 0) & (v < NHALF)
                didx[pl.ds(k * 16, 16)] = jnp.where(inh, v, NHALF)

            gcopy.wait()

            def l2(j, c):
                exj = exbuf[pl.ds(j, 16)][0]
                for k in range(DH // 16):
                    rows_v[j, pl.ds(k * 16, 16)] = rows_v[j, pl.ds(k * 16, 16)] * exj
                return c
            lax.fori_loop(0, CHUNK, l2, 0, unroll=False)

            pltpu.sync_copy(rows_v, acc_sh.at[didx], add=True)
            return carry

        lax.fori_loop(0, NCHUNK, chunk_body, 0, unroll=False)
        plsc.subcore_barrier()

        for sb in range(10):
            row0 = node0 + sb * 32
            pltpu.sync_copy(acc_sh.at[pl.ds(row0, 32)], epi_v)

            def nbody(rr, c):
                inv = dflat[pl.ds(half * 320 + sb * 32 + rr, 16)][0]
                for k in range(DH // 16):
                    v = epi_v[rr, pl.ds(k * 16, 16)] * inv + bias_v[pl.ds(k * 16, 16)]
                    epi_v[rr, pl.ds(k * 16, 16)] = jnp.maximum(v, 0.0)
                return c
            lax.fori_loop(0, 32, nbody, 0, unroll=False)
            pltpu.sync_copy(
                epi_v,
                out_hbm.at[pl.ds(hbase + row0, 32), pl.ds(core * DH, DH)])
        if half == 0:
            plsc.subcore_barrier()


def _accum_sc(hs, srcp, dstp, ex, dps, bias2):
    mesh = plsc.VectorSubcoreMesh(core_axis_name="c", subcore_axis_name="s")
    kern = functools.partial(
        pl.kernel,
        mesh=mesh,
        out_type=jax.ShapeDtypeStruct((N_PAD, D_HID), jnp.float32),
        scratch_types=[
            pltpu.VMEM((CHUNK + 16,), jnp.int32),
            pltpu.VMEM((CHUNK,), jnp.int32),
            pltpu.VMEM((CHUNK + 16,), jnp.int32),
            pltpu.VMEM((CHUNK,), jnp.int32),
            pltpu.VMEM((CHUNK + 16,), jnp.float32),
            pltpu.VMEM((CHUNK, DH), jnp.float32),
            pltpu.VMEM((32, DH), jnp.float32),
            pltpu.VMEM((24, 16), jnp.float32),
            pltpu.VMEM((40, 16), jnp.float32),
            pltpu.VMEM((640 + 16,), jnp.float32),
            pltpu.VMEM((DH,), jnp.float32),
            pltpu.SemaphoreType.DMA,
            pltpu.VMEM_SHARED((NHALF + 8, DH), jnp.float32),
        ],
    )(_accum_body)
    return kern(hs, srcp, dstp, ex, dps, bias2)


def _edge_phase_sc(hs, srcp, dstp, asv, adv, mrep, bias2):
    ex, dps = _score_sc(srcp, dstp, asv, adv, mrep)
    return _accum_sc(hs, srcp, dstp, ex, dps, bias2)


def _readout_body(h_ref, batch_ref, pw_ref, pb_ref, rw_ref, rb_ref, out_ref):
    b = batch_ref[...]
    onehot = (b == jax.lax.broadcasted_iota(jnp.int32, (1, N_GRAPHS), 1)).astype(jnp.float32)
    g = jnp.dot(onehot.T, h_ref[...], preferred_element_type=jnp.float32)
    g = jnp.maximum(jnp.dot(g, pw_ref[...], preferred_element_type=jnp.float32) + pb_ref[...], 0.0)
    logits = jnp.dot(g, rw_ref[...], preferred_element_type=jnp.float32) + rb_ref[...]
    col = jax.lax.broadcasted_iota(jnp.int32, logits.shape, 1)
    logits = jnp.where(col < N_CLASS, logits, -jnp.inf)
    m = jnp.max(logits, axis=1, keepdims=True)
    s = logits - m
    lse = jnp.log(jnp.sum(jnp.exp(s), axis=1, keepdims=True))
    out_ref[...] = s - lse


def _readout(h, batch, postW, postb, roW, rob):
    n, d = h.shape
    ro_pad = jnp.zeros((d, 128), jnp.float32).at[:, :N_CLASS].set(roW)
    rb_pad = jnp.zeros((128,), jnp.float32).at[:N_CLASS].set(rob)
    out = pl.pallas_call(
        _readout_body,
        in_specs=[
            pl.BlockSpec((n, d), lambda: (0, 0)),
            pl.BlockSpec((n, 1), lambda: (0, 0)),
            pl.BlockSpec((d, d), lambda: (0, 0)),
            pl.BlockSpec((1, d), lambda: (0, 0)),
            pl.BlockSpec((d, 128), lambda: (0, 0)),
            pl.BlockSpec((1, 128), lambda: (0, 0)),
        ],
        out_specs=pl.BlockSpec((N_GRAPHS, 128), lambda: (0, 0)),
        out_shape=jax.ShapeDtypeStruct((N_GRAPHS, 128), jnp.float32),
    )(h, batch[:, None], postW, postb[None, :], ro_pad, rb_pad[None, :])
    return out[:, :N_CLASS]


def kernel(x, edge_index, batch, W1, a_src1, a_dst1, b1, W2, a_src2, a_dst2, b2,
           W3, a_src3, a_dst3, b3, postW, postb, roW, rob):
    n = x.shape[0]
    loops = jnp.arange(n, dtype=edge_index.dtype)
    padv = jnp.full((E_PAD - N_EDGES_TOT,), N_PAD - 1, edge_index.dtype)
    srcp = jnp.concatenate([edge_index[0], loops, padv])
    dstp = jnp.concatenate([edge_index[1], loops, padv])

    h = jnp.pad(x, ((0, N_PAD - n), (0, 0)))
    for (W, a_s, a_d, b) in ((W1, a_src1, a_dst1, b1), (W2, a_src2, a_dst2, b2),
                             (W3, a_src3, a_dst3, b3)):
        hs, al = _mm_attn(h, W, a_s, a_d)
        asv = al[:, 0]
        adv = al[:, 1]
        m = jnp.maximum(jnp.max(asv[:n]) + jnp.max(adv[:n]), 0.0)
        mrep = jnp.full((16,), m, jnp.float32)
        bias2 = jnp.stack([b[:DH], b[DH:]])
        h = _edge_phase_sc(hs, srcp, dstp, jnp.pad(asv, (0, 16)),
                           jnp.pad(adv, (0, 16)), mrep, bias2)

    return _readout(h[:n], batch, postW, postb, roW, rob)


# speedup vs baseline: 5.3705x; 1.2862x over previous
import functools

import jax
import jax.numpy as jnp
from jax import lax
from jax.experimental import pallas as pl
from jax.experimental.pallas import tpu as pltpu
from jax.experimental.pallas import tpu_sc as plsc

N_NODES = 10000
N_PAD = 10240
NHALF = 5120
N_GRAPHS = 128
N_CLASS = 10
NEG_SLOPE = 0.2
D_HID = 256
DH = 128
N_EDGES_TOT = 330000
CHUNK = 128
EDGES_PER_TILE = 20736
E_PAD = 16 * EDGES_PER_TILE
NCHUNK = EDGES_PER_TILE // CHUNK

_BLK = 1024


def _mm_attn_body(x_ref, w_ref, apad_ref, hs_ref, al_ref):
    h = jnp.dot(x_ref[...], w_ref[...], preferred_element_type=jnp.float32)
    hs_ref[0] = h[:, :DH]
    hs_ref[1] = h[:, DH:]
    al_ref[...] = jnp.dot(h, apad_ref[...], preferred_element_type=jnp.float32)


def _mm_attn(x, W, a_src, a_dst):
    n, d_in = x.shape
    apad = jnp.zeros((D_HID, 128), jnp.float32)
    apad = apad.at[:, 0].set(a_src).at[:, 1].set(a_dst)
    hs, al = pl.pallas_call(
        _mm_attn_body,
        grid=(n // _BLK,),
        in_specs=[
            pl.BlockSpec((_BLK, d_in), lambda i: (i, 0)),
            pl.BlockSpec((d_in, D_HID), lambda i: (0, 0)),
            pl.BlockSpec((D_HID, 128), lambda i: (0, 0)),
        ],
        out_specs=[
            pl.BlockSpec((2, _BLK, DH), lambda i: (0, i, 0)),
            pl.BlockSpec((_BLK, 128), lambda i: (i, 0)),
        ],
        out_shape=[
            jax.ShapeDtypeStruct((2, n, DH), jnp.float32),
            jax.ShapeDtypeStruct((n, 128), jnp.float32),
        ],
    )(x, W, apad)
    return hs.reshape(2 * n, DH), al


EPT_A = E_PAD // 32
NCH_A = EPT_A // CHUNK


def _score_body(srcp_hbm, dstp_hbm, asv_hbm, adv_hbm, mrep_hbm,
                ex_hbm, dps_hbm,
                astab, adtab, sbuf2, dbuf2, ebuf, exw, dp_v, mv):
    core = lax.axis_index("c")
    wid = lax.axis_index("s")
    iota = lax.iota(jnp.int32, 16)

    pltpu.sync_copy(asv_hbm, astab)
    pltpu.sync_copy(adv_hbm, adtab)
    pltpu.sync_copy(mrep_hbm, mv)
    m_s = mv[...][0]

    for i in range(640):
        dp_v[i, :] = jnp.zeros((16,), jnp.float32)

    tid = core * 16 + wid
    ebase = tid * EPT_A

    def chunk_body(ch, carry):
        base = ebase + ch * CHUNK
        pltpu.sync_copy(srcp_hbm.at[pl.ds(base, CHUNK)], sbuf2.at[pl.ds(0, CHUNK)])
        pltpu.sync_copy(dstp_hbm.at[pl.ds(base, CHUNK)], dbuf2.at[pl.ds(0, CHUNK)])

        def l1(j, c):
            s = sbuf2[pl.ds(j, 16)][0]
            d = dbuf2[pl.ds(j, 16)][0]
            e = astab[pl.ds(s, 16)][0] + adtab[pl.ds(d, 16)][0]
            e = jnp.where(e > 0, e, e * NEG_SLOPE) - m_s
            ebuf[j >> 4, :] = jnp.where(iota == (j & 15), e, ebuf[j >> 4, :])
            return c
        lax.fori_loop(0, CHUNK, l1, 0, unroll=4)
        for i in range(CHUNK // 16):
            exw[pl.ds(i * 16, 16)] = jnp.exp(ebuf[i, :])

        def l2(j, c):
            exj = exw[pl.ds(j, 16)][0]
            dj = dbuf2[pl.ds(j, 16)][0]
            dp_v[dj >> 4, :] = dp_v[dj >> 4, :] + jnp.where(iota == (dj & 15), exj, 0.0)
            return c
        lax.fori_loop(0, CHUNK, l2, 0, unroll=4)

        pltpu.sync_copy(exw.at[pl.ds(0, CHUNK)], ex_hbm.at[pl.ds(base, CHUNK)])
        return carry

    lax.fori_loop(0, NCH_A, chunk_body, 0, unroll=False)
    pltpu.sync_copy(dp_v, dps_hbm.at[core, wid])


def _score_sc(srcp, dstp, asv, adv, mrep):
    mesh = plsc.VectorSubcoreMesh(core_axis_name="c", subcore_axis_name="s")
    kern = functools.partial(
        pl.kernel,
        mesh=mesh,
        out_type=(jax.ShapeDtypeStruct((E_PAD,), jnp.float32),
                  jax.ShapeDtypeStruct((2, 16, 640, 16), jnp.float32)),
        scratch_types=[
            pltpu.VMEM((N_PAD + 16,), jnp.float32),
            pltpu.VMEM((N_PAD + 16,), jnp.float32),
            pltpu.VMEM((CHUNK + 16,), jnp.int32),
            pltpu.VMEM((CHUNK + 16,), jnp.int32),
            pltpu.VMEM((CHUNK // 16, 16), jnp.float32),
            pltpu.VMEM((CHUNK + 16,), jnp.float32),
            pltpu.VMEM((640, 16), jnp.float32),
            pltpu.VMEM((16,), jnp.float32),
        ],
    )(_score_body)
    return kern(srcp, dstp, asv, adv, mrep)


def _accum_body(hs_hbm, srcp_hbm, dstp_hbm, ex_hbm, dps_hbm, bias_hbm,
                out_hbm,
                sbuf2, sidx, dbuf2, didx, exbuf, rows_v, epi_v,
                dtmp, dacc, dflat, bias_v, gsem,
                acc_sh):
    core = lax.axis_index("c")
    wid = lax.axis_index("s")

    pltpu.sync_copy(bias_hbm.at[core], bias_v)

    for i in range(40):
        dacc[i, :] = jnp.zeros((16,), jnp.float32)
    for c2 in range(2):
        for p in range(16):
            for h2 in range(2):
                start = h2 * 320 + wid * 20
                off = start & 7
                a0 = pl.multiple_of(start - off, 8)
                pltpu.sync_copy(dps_hbm.at[c2, p, pl.ds(a0, 24)], dtmp)
                for i in range(20):
                    dacc[h2 * 20 + i, :] = dacc[h2 * 20 + i, :] + dtmp[off + i, :]
    one = jnp.ones((16,), jnp.float32)
    for i in range(40):
        dflat[pl.ds(i * 16, 16)] = one / (dacc[i, :] + 1e-16)

    ebase = wid * EDGES_PER_TILE
    node0 = wid * 320

    for half in range(2):
        hbase = half * NHALF

        for k in range(DH // 16):
            epi_v[:, pl.ds(k * 16, 16)] = jnp.zeros((32, 16), jnp.float32)
        for sb in range(10):
            pltpu.sync_copy(epi_v, acc_sh.at[pl.ds(node0 + sb * 32, 32)])
        @pl.when(wid == 0)
        def _():
            pltpu.sync_copy(epi_v.at[pl.ds(0, 8)], acc_sh.at[pl.ds(NHALF, 8)])
        plsc.subcore_barrier()

        def chunk_body(ch, carry):
            base = ebase + ch * CHUNK
            pltpu.sync_copy(srcp_hbm.at[pl.ds(base, CHUNK)], sbuf2.at[pl.ds(0, CHUNK)])
            pltpu.sync_copy(dstp_hbm.at[pl.ds(base, CHUNK)], dbuf2.at[pl.ds(0, CHUNK)])
            pltpu.sync_copy(ex_hbm.at[pl.ds(base, CHUNK)], exbuf.at[pl.ds(0, CHUNK)])
            off = core * N_PAD
            for k in range(CHUNK // 16):
                sidx[pl.ds(k * 16, 16)] = sbuf2[pl.ds(k * 16, 16)] + off
            gcopy = pltpu.async_copy(hs_hbm.at[sidx], rows_v, gsem)
            for k in range(CHUNK // 16):
                v = dbuf2[pl.ds(k * 16, 16)] - hbase
                inh = (v >= 0) & (v < NHALF)
                didx[pl.ds(k * 16, 16)] = jnp.where(inh, v, NHALF)

            gcopy.wait()

            def l2(j, c):
                exj = exbuf[pl.ds(j, 16)][0]
                for k in range(DH // 16):
                    rows_v[j, pl.ds(k * 16, 16)] = rows_v[j, pl.ds(k * 16, 16)] * exj
                return c
            lax.fori_loop(0, CHUNK, l2, 0, unroll=4)

            pltpu.sync_copy(rows_v, acc_sh.at[didx], add=True)
            return carry

        lax.fori_loop(0, NCHUNK, chunk_body, 0, unroll=False)
        plsc.subcore_barrier()

        for sb in range(10):
            row0 = node0 + sb * 32
            pltpu.sync_copy(acc_sh.at[pl.ds(row0, 32)], epi_v)

            def nbody(rr, c):
                inv = dflat[pl.ds(half * 320 + sb * 32 + rr, 16)][0]
                for k in range(DH // 16):
                    v = epi_v[rr, pl.ds(k * 16, 16)] * inv + bias_v[pl.ds(k * 16, 16)]
                    epi_v[rr, pl.ds(k * 16, 16)] = jnp.maximum(v, 0.0)
                return c
            lax.fori_loop(0, 32, nbody, 0, unroll=False)
            pltpu.sync_copy(
                epi_v,
                out_hbm.at[pl.ds(hbase + row0, 32), pl.ds(core * DH, DH)])
        if half == 0:
            plsc.subcore_barrier()


def _accum_sc(hs, srcp, dstp, ex, dps, bias2):
    mesh = plsc.VectorSubcoreMesh(core_axis_name="c", subcore_axis_name="s")
    kern = functools.partial(
        pl.kernel,
        mesh=mesh,
        out_type=jax.ShapeDtypeStruct((N_PAD, D_HID), jnp.float32),
        scratch_types=[
            pltpu.VMEM((CHUNK + 16,), jnp.int32),
            pltpu.VMEM((CHUNK,), jnp.int32),
            pltpu.VMEM((CHUNK + 16,), jnp.int32),
            pltpu.VMEM((CHUNK,), jnp.int32),
            pltpu.VMEM((CHUNK + 16,), jnp.float32),
            pltpu.VMEM((CHUNK, DH), jnp.float32),
            pltpu.VMEM((32, DH), jnp.float32),
            pltpu.VMEM((24, 16), jnp.float32),
            pltpu.VMEM((40, 16), jnp.float32),
            pltpu.VMEM((640 + 16,), jnp.float32),
            pltpu.VMEM((DH,), jnp.float32),
            pltpu.SemaphoreType.DMA,
            pltpu.VMEM_SHARED((NHALF + 8, DH), jnp.float32),
        ],
    )(_accum_body)
    return kern(hs, srcp, dstp, ex, dps, bias2)


def _edge_phase_sc(hs, srcp, dstp, asv, adv, mrep, bias2):
    ex, dps = _score_sc(srcp, dstp, asv, adv, mrep)
    return _accum_sc(hs, srcp, dstp, ex, dps, bias2)


def _readout_body(h_ref, batch_ref, pw_ref, pb_ref, rw_ref, rb_ref, out_ref):
    b = batch_ref[...]
    onehot = (b == jax.lax.broadcasted_iota(jnp.int32, (1, N_GRAPHS), 1)).astype(jnp.float32)
    g = jnp.dot(onehot.T, h_ref[...], preferred_element_type=jnp.float32)
    g = jnp.maximum(jnp.dot(g, pw_ref[...], preferred_element_type=jnp.float32) + pb_ref[...], 0.0)
    logits = jnp.dot(g, rw_ref[...], preferred_element_type=jnp.float32) + rb_ref[...]
    col = jax.lax.broadcasted_iota(jnp.int32, logits.shape, 1)
    logits = jnp.where(col < N_CLASS, logits, -jnp.inf)
    m = jnp.max(logits, axis=1, keepdims=True)
    s = logits - m
    lse = jnp.log(jnp.sum(jnp.exp(s), axis=1, keepdims=True))
    out_ref[...] = s - lse


def _readout(h, batch, postW, postb, roW, rob):
    n, d = h.shape
    ro_pad = jnp.zeros((d, 128), jnp.float32).at[:, :N_CLASS].set(roW)
    rb_pad = jnp.zeros((128,), jnp.float32).at[:N_CLASS].set(rob)
    out = pl.pallas_call(
        _readout_body,
        in_specs=[
            pl.BlockSpec((n, d), lambda: (0, 0)),
            pl.BlockSpec((n, 1), lambda: (0, 0)),
            pl.BlockSpec((d, d), lambda: (0, 0)),
            pl.BlockSpec((1, d), lambda: (0, 0)),
            pl.BlockSpec((d, 128), lambda: (0, 0)),
            pl.BlockSpec((1, 128), lambda: (0, 0)),
        ],
        out_specs=pl.BlockSpec((N_GRAPHS, 128), lambda: (0, 0)),
        out_shape=jax.ShapeDtypeStruct((N_GRAPHS, 128), jnp.float32),
    )(h, batch[:, None], postW, postb[None, :], ro_pad, rb_pad[None, :])
    return out[:, :N_CLASS]


def kernel(x, edge_index, batch, W1, a_src1, a_dst1, b1, W2, a_src2, a_dst2, b2,
           W3, a_src3, a_dst3, b3, postW, postb, roW, rob):
    n = x.shape[0]
    loops = jnp.arange(n, dtype=edge_index.dtype)
    padv = jnp.full((E_PAD - N_EDGES_TOT,), N_PAD - 1, edge_index.dtype)
    srcp = jnp.concatenate([edge_index[0], loops, padv])
    dstp = jnp.concatenate([edge_index[1], loops, padv])

    h = jnp.pad(x, ((0, N_PAD - n), (0, 0)))
    for (W, a_s, a_d, b) in ((W1, a_src1, a_dst1, b1), (W2, a_src2, a_dst2, b2),
                             (W3, a_src3, a_dst3, b3)):
        hs, al = _mm_attn(h, W, a_s, a_d)
        asv = al[:, 0]
        adv = al[:, 1]
        m = jnp.maximum(jnp.max(asv[:n]) + jnp.max(adv[:n]), 0.0)
        mrep = jnp.full((16,), m, jnp.float32)
        bias2 = jnp.stack([b[:DH], b[DH:]])
        h = _edge_phase_sc(hs, srcp, dstp, jnp.pad(asv, (0, 16)),
                           jnp.pad(adv, (0, 16)), mrep, bias2)

    return _readout(h[:n], batch, postW, postb, roW, rob)


# speedup vs baseline: 5.9614x; 1.1100x over previous
import functools

import jax
import jax.numpy as jnp
from jax import lax
from jax.experimental import pallas as pl
from jax.experimental.pallas import tpu as pltpu
from jax.experimental.pallas import tpu_sc as plsc

N_NODES = 10000
N_PAD = 10240
NHALF = 5120
N_GRAPHS = 128
N_CLASS = 10
NEG_SLOPE = 0.2
D_HID = 256
DH = 128
N_EDGES_TOT = 330000
CHUNK = 128
EDGES_PER_TILE = 20736
E_PAD = 16 * EDGES_PER_TILE
NCHUNK = EDGES_PER_TILE // CHUNK

_BLK = 1024


def _mm_attn_body(x_ref, w_ref, apad_ref, hs_ref, al_ref):
    h = jnp.dot(x_ref[...], w_ref[...], preferred_element_type=jnp.float32)
    hs_ref[0] = h[:, :DH]
    hs_ref[1] = h[:, DH:]
    al_ref[...] = jnp.dot(h, apad_ref[...], preferred_element_type=jnp.float32)


def _mm_attn(x, W, a_src, a_dst):
    n, d_in = x.shape
    apad = jnp.zeros((D_HID, 128), jnp.float32)
    apad = apad.at[:, 0].set(a_src).at[:, 1].set(a_dst)
    hs, al = pl.pallas_call(
        _mm_attn_body,
        grid=(n // _BLK,),
        in_specs=[
            pl.BlockSpec((_BLK, d_in), lambda i: (i, 0)),
            pl.BlockSpec((d_in, D_HID), lambda i: (0, 0)),
            pl.BlockSpec((D_HID, 128), lambda i: (0, 0)),
        ],
        out_specs=[
            pl.BlockSpec((2, _BLK, DH), lambda i: (0, i, 0)),
            pl.BlockSpec((_BLK, 128), lambda i: (i, 0)),
        ],
        out_shape=[
            jax.ShapeDtypeStruct((2, n, DH), jnp.float32),
            jax.ShapeDtypeStruct((n, 128), jnp.float32),
        ],
    )(x, W, apad)
    return hs.reshape(2 * n, DH), al


EPT_A = E_PAD // 32
NCH_A = EPT_A // CHUNK


def _score_body(srcp_hbm, dstp_hbm, asv_hbm, adv_hbm, mrep_hbm,
                ex_hbm, dps_hbm,
                astab, adtab, sbuf2, dbuf2, ebuf, exw, dp_v, mv):
    core = lax.axis_index("c")
    wid = lax.axis_index("s")
    iota = lax.iota(jnp.int32, 16)

    pltpu.sync_copy(asv_hbm, astab)
    pltpu.sync_copy(adv_hbm, adtab)
    pltpu.sync_copy(mrep_hbm, mv)
    m_s = mv[...][0]

    for i in range(640):
        dp_v[i, :] = jnp.zeros((16,), jnp.float32)

    tid = core * 16 + wid
    ebase = tid * EPT_A

    def chunk_body(ch, carry):
        base = ebase + ch * CHUNK
        pltpu.sync_copy(srcp_hbm.at[pl.ds(base, CHUNK)], sbuf2.at[pl.ds(0, CHUNK)])
        pltpu.sync_copy(dstp_hbm.at[pl.ds(base, CHUNK)], dbuf2.at[pl.ds(0, CHUNK)])

        def l1(j, c):
            s = sbuf2[pl.ds(j, 16)][0]
            d = dbuf2[pl.ds(j, 16)][0]
            e = astab[pl.ds(s, 16)][0] + adtab[pl.ds(d, 16)][0]
            e = jnp.where(e > 0, e, e * NEG_SLOPE) - m_s
            ebuf[j >> 4, :] = jnp.where(iota == (j & 15), e, ebuf[j >> 4, :])
            return c
        lax.fori_loop(0, CHUNK, l1, 0, unroll=4)
        for i in range(CHUNK // 16):
            exw[pl.ds(i * 16, 16)] = jnp.exp(ebuf[i, :])

        def l2(j, c):
            exj = exw[pl.ds(j, 16)][0]
            dj = dbuf2[pl.ds(j, 16)][0]
            dp_v[dj >> 4, :] = dp_v[dj >> 4, :] + jnp.where(iota == (dj & 15), exj, 0.0)
            return c
        lax.fori_loop(0, CHUNK, l2, 0, unroll=4)

        pltpu.sync_copy(exw.at[pl.ds(0, CHUNK)], ex_hbm.at[pl.ds(base, CHUNK)])
        return carry

    lax.fori_loop(0, NCH_A, chunk_body, 0, unroll=False)
    pltpu.sync_copy(dp_v, dps_hbm.at[core, wid])


def _score_sc(srcp, dstp, asv, adv, mrep):
    mesh = plsc.VectorSubcoreMesh(core_axis_name="c", subcore_axis_name="s")
    kern = functools.partial(
        pl.kernel,
        mesh=mesh,
        out_type=(jax.ShapeDtypeStruct((E_PAD,), jnp.float32),
                  jax.ShapeDtypeStruct((2, 16, 640, 16), jnp.float32)),
        scratch_types=[
            pltpu.VMEM((N_PAD + 16,), jnp.float32),
            pltpu.VMEM((N_PAD + 16,), jnp.float32),
            pltpu.VMEM((CHUNK + 16,), jnp.int32),
            pltpu.VMEM((CHUNK + 16,), jnp.int32),
            pltpu.VMEM((CHUNK // 16, 16), jnp.float32),
            pltpu.VMEM((CHUNK + 16,), jnp.float32),
            pltpu.VMEM((640, 16), jnp.float32),
            pltpu.VMEM((16,), jnp.float32),
        ],
    )(_score_body)
    return kern(srcp, dstp, asv, adv, mrep)


def _accum_body(hs_hbm, srcp_hbm, dstp_hbm, ex_hbm, dps_hbm, bias_hbm,
                out_hbm, scl_hbm,
                sbuf2, sidx, dbuf2, didx, exbuf, rows_v, epi_v,
                dtmp, dacc, dflat, bias_v, gsem,
                acc_sh):
    core = lax.axis_index("c")
    wid = lax.axis_index("s")

    pltpu.sync_copy(bias_hbm.at[core], bias_v)

    for i in range(40):
        dacc[i, :] = jnp.zeros((16,), jnp.float32)
    for c2 in range(2):
        for p in range(16):
            for h2 in range(2):
                start = h2 * 320 + wid * 20
                off = start & 7
                a0 = pl.multiple_of(start - off, 8)
                pltpu.sync_copy(dps_hbm.at[c2, p, pl.ds(a0, 24)], dtmp)
                for i in range(20):
                    dacc[h2 * 20 + i, :] = dacc[h2 * 20 + i, :] + dtmp[off + i, :]
    one = jnp.ones((16,), jnp.float32)
    for i in range(40):
        dflat[pl.ds(i * 16, 16)] = one / (dacc[i, :] + 1e-16)

    ebase = wid * EDGES_PER_TILE
    node0 = wid * 320

    for half in range(2):
        hbase = half * NHALF

        for k in range(DH // 16):
            epi_v[:, pl.ds(k * 16, 16)] = jnp.zeros((32, 16), jnp.float32)
        for sb in range(10):
            pltpu.sync_copy(epi_v, acc_sh.at[pl.ds(node0 + sb * 32, 32)])
        @pl.when(wid == 0)
        def _():
            pltpu.sync_copy(epi_v.at[pl.ds(0, 8)], acc_sh.at[pl.ds(NHALF, 8)])
        plsc.subcore_barrier()

        def chunk_body(ch, carry):
            base = ebase + ch * CHUNK
            pltpu.sync_copy(dstp_hbm.at[pl.ds(base, CHUNK)], dbuf2.at[pl.ds(0, CHUNK)])
            if half == 0:
                pltpu.sync_copy(srcp_hbm.at[pl.ds(base, CHUNK)], sbuf2.at[pl.ds(0, CHUNK)])
                pltpu.sync_copy(ex_hbm.at[pl.ds(base, CHUNK)], exbuf.at[pl.ds(0, CHUNK)])
                off = core * N_PAD
                for k in range(CHUNK // 16):
                    sidx[pl.ds(k * 16, 16)] = sbuf2[pl.ds(k * 16, 16)] + off
                gcopy = pltpu.async_copy(hs_hbm.at[sidx], rows_v, gsem)
            else:
                gcopy = pltpu.async_copy(scl_hbm.at[core, pl.ds(base, CHUNK)],
                                         rows_v, gsem)
            for k in range(CHUNK // 16):
                v = dbuf2[pl.ds(k * 16, 16)] - hbase
                inh = (v >= 0) & (v < NHALF)
                didx[pl.ds(k * 16, 16)] = jnp.where(inh, v, NHALF)

            gcopy.wait()

            if half == 0:
                def l2(j, c):
                    exj = exbuf[pl.ds(j, 16)][0]
                    for k in range(DH // 16):
                        rows_v[j, pl.ds(k * 16, 16)] = rows_v[j, pl.ds(k * 16, 16)] * exj
                    return c
                lax.fori_loop(0, CHUNK, l2, 0, unroll=4)
                pltpu.sync_copy(rows_v, scl_hbm.at[core, pl.ds(base, CHUNK)])

            pltpu.sync_copy(rows_v, acc_sh.at[didx], add=True)
            return carry

        lax.fori_loop(0, NCHUNK, chunk_body, 0, unroll=False)
        plsc.subcore_barrier()

        for sb in range(10):
            row0 = node0 + sb * 32
            pltpu.sync_copy(acc_sh.at[pl.ds(row0, 32)], epi_v)

            def nbody(rr, c):
                inv = dflat[pl.ds(half * 320 + sb * 32 + rr, 16)][0]
                for k in range(DH // 16):
                    v = epi_v[rr, pl.ds(k * 16, 16)] * inv + bias_v[pl.ds(k * 16, 16)]
                    epi_v[rr, pl.ds(k * 16, 16)] = jnp.maximum(v, 0.0)
                return c
            lax.fori_loop(0, 32, nbody, 0, unroll=False)
            pltpu.sync_copy(
                epi_v,
                out_hbm.at[pl.ds(hbase + row0, 32), pl.ds(core * DH, DH)])
        if half == 0:
            plsc.subcore_barrier()


def _accum_sc(hs, srcp, dstp, ex, dps, bias2):
    mesh = plsc.VectorSubcoreMesh(core_axis_name="c", subcore_axis_name="s")
    kern = functools.partial(
        pl.kernel,
        mesh=mesh,
        out_type=(jax.ShapeDtypeStruct((N_PAD, D_HID), jnp.float32),
                  jax.ShapeDtypeStruct((2, E_PAD, DH), jnp.float32)),
        scratch_types=[
            pltpu.VMEM((CHUNK + 16,), jnp.int32),
            pltpu.VMEM((CHUNK,), jnp.int32),
            pltpu.VMEM((CHUNK + 16,), jnp.int32),
            pltpu.VMEM((CHUNK,), jnp.int32),
            pltpu.VMEM((CHUNK + 16,), jnp.float32),
            pltpu.VMEM((CHUNK, DH), jnp.float32),
            pltpu.VMEM((32, DH), jnp.float32),
            pltpu.VMEM((24, 16), jnp.float32),
            pltpu.VMEM((40, 16), jnp.float32),
            pltpu.VMEM((640 + 16,), jnp.float32),
            pltpu.VMEM((DH,), jnp.float32),
            pltpu.SemaphoreType.DMA,
            pltpu.VMEM_SHARED((NHALF + 8, DH), jnp.float32),
        ],
    )(_accum_body)
    out, _ = kern(hs, srcp, dstp, ex, dps, bias2)
    return out


def _edge_phase_sc(hs, srcp, dstp, asv, adv, mrep, bias2):
    ex, dps = _score_sc(srcp, dstp, asv, adv, mrep)
    return _accum_sc(hs, srcp, dstp, ex, dps, bias2)


def _readout_body(h_ref, batch_ref, pw_ref, pb_ref, rw_ref, rb_ref, out_ref):
    b = batch_ref[...]
    onehot = (b == jax.lax.broadcasted_iota(jnp.int32, (1, N_GRAPHS), 1)).astype(jnp.float32)
    g = jnp.dot(onehot.T, h_ref[...], preferred_element_type=jnp.float32)
    g = jnp.maximum(jnp.dot(g, pw_ref[...], preferred_element_type=jnp.float32) + pb_ref[...], 0.0)
    logits = jnp.dot(g, rw_ref[...], preferred_element_type=jnp.float32) + rb_ref[...]
    col = jax.lax.broadcasted_iota(jnp.int32, logits.shape, 1)
    logits = jnp.where(col < N_CLASS, logits, -jnp.inf)
    m = jnp.max(logits, axis=1, keepdims=True)
    s = logits - m
    lse = jnp.log(jnp.sum(jnp.exp(s), axis=1, keepdims=True))
    out_ref[...] = s - lse


def _readout(h, batch, postW, postb, roW, rob):
    n, d = h.shape
    ro_pad = jnp.zeros((d, 128), jnp.float32).at[:, :N_CLASS].set(roW)
    rb_pad = jnp.zeros((128,), jnp.float32).at[:N_CLASS].set(rob)
    out = pl.pallas_call(
        _readout_body,
        in_specs=[
            pl.BlockSpec((n, d), lambda: (0, 0)),
            pl.BlockSpec((n, 1), lambda: (0, 0)),
            pl.BlockSpec((d, d), lambda: (0, 0)),
            pl.BlockSpec((1, d), lambda: (0, 0)),
            pl.BlockSpec((d, 128), lambda: (0, 0)),
            pl.BlockSpec((1, 128), lambda: (0, 0)),
        ],
        out_specs=pl.BlockSpec((N_GRAPHS, 128), lambda: (0, 0)),
        out_shape=jax.ShapeDtypeStruct((N_GRAPHS, 128), jnp.float32),
    )(h, batch[:, None], postW, postb[None, :], ro_pad, rb_pad[None, :])
    return out[:, :N_CLASS]


def kernel(x, edge_index, batch, W1, a_src1, a_dst1, b1, W2, a_src2, a_dst2, b2,
           W3, a_src3, a_dst3, b3, postW, postb, roW, rob):
    n = x.shape[0]
    loops = jnp.arange(n, dtype=edge_index.dtype)
    padv = jnp.full((E_PAD - N_EDGES_TOT,), N_PAD - 1, edge_index.dtype)
    srcp = jnp.concatenate([edge_index[0], loops, padv])
    dstp = jnp.concatenate([edge_index[1], loops, padv])

    h = jnp.pad(x, ((0, N_PAD - n), (0, 0)))
    for (W, a_s, a_d, b) in ((W1, a_src1, a_dst1, b1), (W2, a_src2, a_dst2, b2),
                             (W3, a_src3, a_dst3, b3)):
        hs, al = _mm_attn(h, W, a_s, a_d)
        asv = al[:, 0]
        adv = al[:, 1]
        m = jnp.maximum(jnp.max(asv[:n]) + jnp.max(adv[:n]), 0.0)
        mrep = jnp.full((16,), m, jnp.float32)
        bias2 = jnp.stack([b[:DH], b[DH:]])
        h = _edge_phase_sc(hs, srcp, dstp, jnp.pad(asv, (0, 16)),
                           jnp.pad(adv, (0, 16)), mrep, bias2)

    return _readout(h[:n], batch, postW, postb, roW, rob)


# speedup vs baseline: 5.9674x; 1.0010x over previous
import functools

import jax
import jax.numpy as jnp
from jax import lax
from jax.experimental import pallas as pl
from jax.experimental.pallas import tpu as pltpu
from jax.experimental.pallas import tpu_sc as plsc

N_NODES = 10000
N_PAD = 10240
NHALF = 5120
N_GRAPHS = 128
N_CLASS = 10
NEG_SLOPE = 0.2
D_HID = 256
DH = 128
N_EDGES_TOT = 330000
CHUNK = 128
EDGES_PER_TILE = 20736
E_PAD = 16 * EDGES_PER_TILE
NCHUNK = EDGES_PER_TILE // CHUNK

_BLK = 1024


def _mm_attn_body(x_ref, w_ref, apad_ref, hs_ref, al_ref):
    h = jnp.dot(x_ref[...], w_ref[...], preferred_element_type=jnp.float32)
    hs_ref[0] = h[:, :DH]
    hs_ref[1] = h[:, DH:]
    al_ref[...] = jnp.dot(h, apad_ref[...], preferred_element_type=jnp.float32)


def _mm_attn(x, W, a_src, a_dst):
    n, d_in = x.shape
    apad = jnp.zeros((D_HID, 128), jnp.float32)
    apad = apad.at[:, 0].set(a_src).at[:, 1].set(a_dst)
    hs, al = pl.pallas_call(
        _mm_attn_body,
        grid=(n // _BLK,),
        in_specs=[
            pl.BlockSpec((_BLK, d_in), lambda i: (i, 0)),
            pl.BlockSpec((d_in, D_HID), lambda i: (0, 0)),
            pl.BlockSpec((D_HID, 128), lambda i: (0, 0)),
        ],
        out_specs=[
            pl.BlockSpec((2, _BLK, DH), lambda i: (0, i, 0)),
            pl.BlockSpec((_BLK, 128), lambda i: (i, 0)),
        ],
        out_shape=[
            jax.ShapeDtypeStruct((2, n, DH), jnp.float32),
            jax.ShapeDtypeStruct((n, 128), jnp.float32),
        ],
    )(x, W, apad)
    return hs.reshape(2 * n, DH), al


EPT_A = E_PAD // 32
NCH_A = EPT_A // CHUNK


def _score_body(srcp_hbm, dstp_hbm, asv_hbm, adv_hbm, mrep_hbm,
                ex_hbm, dps_hbm,
                astab, adtab, sbuf2, dbuf2, ebuf, exw, dp_v, mv):
    core = lax.axis_index("c")
    wid = lax.axis_index("s")
    iota = lax.iota(jnp.int32, 16)

    pltpu.sync_copy(asv_hbm, astab)
    pltpu.sync_copy(adv_hbm, adtab)
    pltpu.sync_copy(mrep_hbm, mv)
    m_s = mv[...][0]

    for i in range(640):
        dp_v[i, :] = jnp.zeros((16,), jnp.float32)

    tid = core * 16 + wid
    ebase = tid * EPT_A

    def chunk_body(ch, carry):
        base = ebase + ch * CHUNK
        pltpu.sync_copy(srcp_hbm.at[pl.ds(base, CHUNK)], sbuf2.at[pl.ds(0, CHUNK)])
        pltpu.sync_copy(dstp_hbm.at[pl.ds(base, CHUNK)], dbuf2.at[pl.ds(0, CHUNK)])

        def l1(j, c):
            s = sbuf2[pl.ds(j, 16)][0]
            d = dbuf2[pl.ds(j, 16)][0]
            e = astab[pl.ds(s, 16)][0] + adtab[pl.ds(d, 16)][0]
            e = jnp.where(e > 0, e, e * NEG_SLOPE) - m_s
            ebuf[j >> 4, :] = jnp.where(iota == (j & 15), e, ebuf[j >> 4, :])
            return c
        lax.fori_loop(0, CHUNK, l1, 0, unroll=4)
        for i in range(CHUNK // 16):
            exw[pl.ds(i * 16, 16)] = jnp.exp(ebuf[i, :])

        def l2(j, c):
            exj = exw[pl.ds(j, 16)][0]
            dj = dbuf2[pl.ds(j, 16)][0]
            dp_v[dj >> 4, :] = dp_v[dj >> 4, :] + jnp.where(iota == (dj & 15), exj, 0.0)
            return c
        lax.fori_loop(0, CHUNK, l2, 0, unroll=4)

        pltpu.sync_copy(exw.at[pl.ds(0, CHUNK)], ex_hbm.at[pl.ds(base, CHUNK)])
        return carry

    lax.fori_loop(0, NCH_A, chunk_body, 0, unroll=False)
    pltpu.sync_copy(dp_v, dps_hbm.at[core, wid])


def _score_sc(srcp, dstp, asv, adv, mrep):
    mesh = plsc.VectorSubcoreMesh(core_axis_name="c", subcore_axis_name="s")
    kern = functools.partial(
        pl.kernel,
        mesh=mesh,
        out_type=(jax.ShapeDtypeStruct((E_PAD,), jnp.float32),
                  jax.ShapeDtypeStruct((2, 16, 640, 16), jnp.float32)),
        scratch_types=[
            pltpu.VMEM((N_PAD + 16,), jnp.float32),
            pltpu.VMEM((N_PAD + 16,), jnp.float32),
            pltpu.VMEM((CHUNK + 16,), jnp.int32),
            pltpu.VMEM((CHUNK + 16,), jnp.int32),
            pltpu.VMEM((CHUNK // 16, 16), jnp.float32),
            pltpu.VMEM((CHUNK + 16,), jnp.float32),
            pltpu.VMEM((640, 16), jnp.float32),
            pltpu.VMEM((16,), jnp.float32),
        ],
    )(_score_body)
    return kern(srcp, dstp, asv, adv, mrep)


def _accum_body(hs_hbm, srcp_hbm, dstp_hbm, ex_hbm, dps_hbm, bias_hbm,
                out_hbm, scl_hbm,
                sbuf2, sidx, dbuf2, didx, exbuf, rows_v, epi_v,
                dtmp, dacc, dflat, bias_v, gsem,
                acc_sh):
    core = lax.axis_index("c")
    wid = lax.axis_index("s")

    pltpu.sync_copy(bias_hbm.at[core], bias_v)

    for i in range(40):
        dacc[i, :] = jnp.zeros((16,), jnp.float32)
    for c2 in range(2):
        for p in range(16):
            for h2 in range(2):
                start = h2 * 320 + wid * 20
                off = start & 7
                a0 = pl.multiple_of(start - off, 8)
                pltpu.sync_copy(dps_hbm.at[c2, p, pl.ds(a0, 24)], dtmp)
                for i in range(20):
                    dacc[h2 * 20 + i, :] = dacc[h2 * 20 + i, :] + dtmp[off + i, :]
    one = jnp.ones((16,), jnp.float32)
    for i in range(40):
        dflat[pl.ds(i * 16, 16)] = one / (dacc[i, :] + 1e-16)

    ebase = wid * EDGES_PER_TILE
    node0 = wid * 320

    for half in range(2):
        hbase = half * NHALF

        for k in range(DH // 16):
            epi_v[:, pl.ds(k * 16, 16)] = jnp.zeros((32, 16), jnp.float32)
        for sb in range(10):
            pltpu.sync_copy(epi_v, acc_sh.at[pl.ds(node0 + sb * 32, 32)])
        @pl.when(wid == 0)
        def _():
            pltpu.sync_copy(epi_v.at[pl.ds(0, 8)], acc_sh.at[pl.ds(NHALF, 8)])
        plsc.subcore_barrier()

        def chunk_body(ch, carry):
            base = ebase + ch * CHUNK
            pltpu.sync_copy(dstp_hbm.at[pl.ds(base, CHUNK)], dbuf2.at[pl.ds(0, CHUNK)])
            if half == 0:
                pltpu.sync_copy(srcp_hbm.at[pl.ds(base, CHUNK)], sbuf2.at[pl.ds(0, CHUNK)])
                pltpu.sync_copy(ex_hbm.at[pl.ds(base, CHUNK)], exbuf.at[pl.ds(0, CHUNK)])
                off = core * N_PAD
                for k in range(CHUNK // 16):
                    sidx[pl.ds(k * 16, 16)] = sbuf2[pl.ds(k * 16, 16)] + off
                gcopy = pltpu.async_copy(hs_hbm.at[sidx], rows_v, gsem)
            else:
                gcopy = pltpu.async_copy(scl_hbm.at[core, pl.ds(base, CHUNK)],
                                         rows_v, gsem)
            for k in range(CHUNK // 16):
                v = dbuf2[pl.ds(k * 16, 16)] - hbase
                inh = (v >= 0) & (v < NHALF)
                didx[pl.ds(k * 16, 16)] = jnp.where(inh, v, NHALF)

            gcopy.wait()

            if half == 0:
                def l2(j, c):
                    exj = exbuf[pl.ds(j, 16)][0]
                    for k in range(DH // 16):
                        rows_v[j, pl.ds(k * 16, 16)] = rows_v[j, pl.ds(k * 16, 16)] * exj
                    return c
                lax.fori_loop(0, CHUNK, l2, 0, unroll=8)
                pltpu.sync_copy(rows_v, scl_hbm.at[core, pl.ds(base, CHUNK)])

            pltpu.sync_copy(rows_v, acc_sh.at[didx], add=True)
            return carry

        lax.fori_loop(0, NCHUNK, chunk_body, 0, unroll=False)
        plsc.subcore_barrier()

        for sb in range(10):
            row0 = node0 + sb * 32
            pltpu.sync_copy(acc_sh.at[pl.ds(row0, 32)], epi_v)

            def nbody(rr, c):
                inv = dflat[pl.ds(half * 320 + sb * 32 + rr, 16)][0]
                for k in range(DH // 16):
                    v = epi_v[rr, pl.ds(k * 16, 16)] * inv + bias_v[pl.ds(k * 16, 16)]
                    epi_v[rr, pl.ds(k * 16, 16)] = jnp.maximum(v, 0.0)
                return c
            lax.fori_loop(0, 32, nbody, 0, unroll=False)
            pltpu.sync_copy(
                epi_v,
                out_hbm.at[pl.ds(hbase + row0, 32), pl.ds(core * DH, DH)])
        if half == 0:
            plsc.subcore_barrier()


def _accum_sc(hs, srcp, dstp, ex, dps, bias2):
    mesh = plsc.VectorSubcoreMesh(core_axis_name="c", subcore_axis_name="s")
    kern = functools.partial(
        pl.kernel,
        mesh=mesh,
        out_type=(jax.ShapeDtypeStruct((N_PAD, D_HID), jnp.float32),
                  jax.ShapeDtypeStruct((2, E_PAD, DH), jnp.float32)),
        scratch_types=[
            pltpu.VMEM((CHUNK + 16,), jnp.int32),
            pltpu.VMEM((CHUNK,), jnp.int32),
            pltpu.VMEM((CHUNK + 16,), jnp.int32),
            pltpu.VMEM((CHUNK,), jnp.int32),
            pltpu.VMEM((CHUNK + 16,), jnp.float32),
            pltpu.VMEM((CHUNK, DH), jnp.float32),
            pltpu.VMEM((32, DH), jnp.float32),
            pltpu.VMEM((24, 16), jnp.float32),
            pltpu.VMEM((40, 16), jnp.float32),
            pltpu.VMEM((640 + 16,), jnp.float32),
            pltpu.VMEM((DH,), jnp.float32),
            pltpu.SemaphoreType.DMA,
            pltpu.VMEM_SHARED((NHALF + 8, DH), jnp.float32),
        ],
    )(_accum_body)
    out, _ = kern(hs, srcp, dstp, ex, dps, bias2)
    return out


def _edge_phase_sc(hs, srcp, dstp, asv, adv, mrep, bias2):
    ex, dps = _score_sc(srcp, dstp, asv, adv, mrep)
    return _accum_sc(hs, srcp, dstp, ex, dps, bias2)


def _readout_body(h_ref, batch_ref, pw_ref, pb_ref, rw_ref, rb_ref, out_ref):
    b = batch_ref[...]
    onehot = (b == jax.lax.broadcasted_iota(jnp.int32, (1, N_GRAPHS), 1)).astype(jnp.float32)
    g = jnp.dot(onehot.T, h_ref[...], preferred_element_type=jnp.float32)
    g = jnp.maximum(jnp.dot(g, pw_ref[...], preferred_element_type=jnp.float32) + pb_ref[...], 0.0)
    logits = jnp.dot(g, rw_ref[...], preferred_element_type=jnp.float32) + rb_ref[...]
    col = jax.lax.broadcasted_iota(jnp.int32, logits.shape, 1)
    logits = jnp.where(col < N_CLASS, logits, -jnp.inf)
    m = jnp.max(logits, axis=1, keepdims=True)
    s = logits - m
    lse = jnp.log(jnp.sum(jnp.exp(s), axis=1, keepdims=True))
    out_ref[...] = s - lse


def _readout(h, batch, postW, postb, roW, rob):
    n, d = h.shape
    ro_pad = jnp.zeros((d, 128), jnp.float32).at[:, :N_CLASS].set(roW)
    rb_pad = jnp.zeros((128,), jnp.float32).at[:N_CLASS].set(rob)
    out = pl.pallas_call(
        _readout_body,
        in_specs=[
            pl.BlockSpec((n, d), lambda: (0, 0)),
            pl.BlockSpec((n, 1), lambda: (0, 0)),
            pl.BlockSpec((d, d), lambda: (0, 0)),
            pl.BlockSpec((1, d), lambda: (0, 0)),
            pl.BlockSpec((d, 128), lambda: (0, 0)),
            pl.BlockSpec((1, 128), lambda: (0, 0)),
        ],
        out_specs=pl.BlockSpec((N_GRAPHS, 128), lambda: (0, 0)),
        out_shape=jax.ShapeDtypeStruct((N_GRAPHS, 128), jnp.float32),
    )(h, batch[:, None], postW, postb[None, :], ro_pad, rb_pad[None, :])
    return out[:, :N_CLASS]


def kernel(x, edge_index, batch, W1, a_src1, a_dst1, b1, W2, a_src2, a_dst2, b2,
           W3, a_src3, a_dst3, b3, postW, postb, roW, rob):
    n = x.shape[0]
    loops = jnp.arange(n, dtype=edge_index.dtype)
    padv = jnp.full((E_PAD - N_EDGES_TOT,), N_PAD - 1, edge_index.dtype)
    srcp = jnp.concatenate([edge_index[0], loops, padv])
    dstp = jnp.concatenate([edge_index[1], loops, padv])

    h = jnp.pad(x, ((0, N_PAD - n), (0, 0)))
    for (W, a_s, a_d, b) in ((W1, a_src1, a_dst1, b1), (W2, a_src2, a_dst2, b2),
                             (W3, a_src3, a_dst3, b3)):
        hs, al = _mm_attn(h, W, a_s, a_d)
        asv = al[:, 0]
        adv = al[:, 1]
        m = jnp.maximum(jnp.max(asv[:n]) + jnp.max(adv[:n]), 0.0)
        mrep = jnp.full((16,), m, jnp.float32)
        bias2 = jnp.stack([b[:DH], b[DH:]])
        h = _edge_phase_sc(hs, srcp, dstp, jnp.pad(asv, (0, 16)),
                           jnp.pad(adv, (0, 16)), mrep, bias2)

    return _readout(h[:n], batch, postW, postb, roW, rob)
